# baseline (speedup 1.0000x reference)
import functools

import jax
import jax.numpy as jnp
from jax import lax
from jax.experimental import pallas as pl
from jax.experimental.pallas import tpu as pltpu
from jax.experimental.pallas import tpu_sc as plsc

_B = 4096
_F = 26
_V = 1000
_E = 64
_NUM = 13
_L = 3
_D = _F * _E + _NUM
_DP = 1792
_H1, _H2, _H3 = 1024, 512, 256

_NC = 2
_NS = 16
_NW = _NC * _NS
_NCH = 4
_BC = _B // _NCH
_RC = _BC * _F
_RPW = _RC // _NW
_CHUNK = 416
_NCHUNK = _RPW // _CHUNK


@functools.lru_cache(maxsize=1)
def _make_sc_gather():
    mesh = plsc.VectorSubcoreMesh(core_axis_name="c", subcore_axis_name="s")

    @functools.partial(
        pl.kernel,
        mesh=mesh,
        out_type=jax.ShapeDtypeStruct((_RC, 128), jnp.float32),
        scratch_types=[
            pltpu.VMEM((_RPW,), jnp.int32),
            pltpu.VMEM((_CHUNK, 128), jnp.float32),
            pltpu.VMEM((_CHUNK, 128), jnp.float32),
            pltpu.SemaphoreType.DMA,
            pltpu.SemaphoreType.DMA,
        ],
    )
    def _sc_gather(tab_hbm, idx_hbm, out_hbm, idx_v, rows0, rows1, sem0, sem1):
        wid = lax.axis_index("s") * _NC + lax.axis_index("c")
        base = wid * _RPW
        pltpu.sync_copy(idx_hbm.at[pl.ds(base, _RPW)], idx_v)
        bufs = (rows0, rows1)
        sems = (sem0, sem1)
        cps = [None, None]
        cps[0] = pltpu.async_copy(tab_hbm.at[idx_v.at[pl.ds(0, _CHUNK)]],
                                  rows0, sem0)
        for c in range(_NCHUNK):
            nxt = (c + 1) % 2
            if c + 1 < _NCHUNK:
                cps[nxt] = pltpu.async_copy(
                    tab_hbm.at[idx_v.at[pl.ds((c + 1) * _CHUNK, _CHUNK)]],
                    bufs[nxt], sems[nxt])
            cps[c % 2].wait()
            pltpu.sync_copy(bufs[c % 2],
                            out_hbm.at[pl.ds(base + c * _CHUNK, _CHUNK)])

    return _sc_gather


_BM = 256


def _dense_block(emb_ref, num_ref, alph_ref, cbias_ref, w1_ref, b1_ref,
                 w2_ref, b2_ref, w3_ref, b3_ref, wct_ref, wcb_ref, bc_ref,
                 out_ref):
    pairs = []
    for p in range(_F // 2):
        a = emb_ref[2 * p]
        b = jnp.roll(emb_ref[2 * p + 1], 64, axis=1)
        pairs.append(a + b)
    x = jnp.concatenate(pairs + [num_ref[...]], axis=1)
    xb = x.astype(jnp.bfloat16)
    cross = x
    for l in range(_L):
        s = jnp.sum(cross * alph_ref[l][None, :], axis=1, keepdims=True)
        cross = cross * (1.0 + s) + cbias_ref[l][None, :]
    h = jnp.dot(xb, w1_ref[...], preferred_element_type=jnp.float32)
    h = jnp.maximum(h + b1_ref[...], 0.0)
    h = jnp.dot(h.astype(jnp.bfloat16), w2_ref[...],
                preferred_element_type=jnp.float32)
    h = jnp.maximum(h + b2_ref[...], 0.0)
    h = (jnp.dot(h.astype(jnp.bfloat16), w3_ref[...],
                 preferred_element_type=jnp.float32) + b3_ref[...])
    z = (jnp.dot(cross, wct_ref[...], preferred_element_type=jnp.float32)
         + jnp.dot(h, wcb_ref[...], preferred_element_type=jnp.float32)
         + bc_ref[0, 0])
    out_ref[...] = jax.nn.sigmoid(z)


def _dense_call(emb3, nump, alph, cbias, w1p, b1, w2, b2, w3, b3, wct, wcb, bc2):
    nblk = _BC // _BM
    full = lambda shape: pl.BlockSpec(shape, lambda i: (0,) * len(shape))
    return pl.pallas_call(
        _dense_block,
        grid=(nblk,),
        in_specs=[
            pl.BlockSpec((_F, _BM, 128), lambda i: (0, i, 0)),
            pl.BlockSpec((_BM, 128), lambda i: (i, 0)),
            full((_L, _DP)),
            full((_L, _DP)),
            full((_DP, _H1)),
            full((1, _H1)),
            full((_H1, _H2)),
            full((1, _H2)),
            full((_H2, _H3)),
            full((1, _H3)),
            full((_DP, 1)),
            full((_H3, 1)),
            full((1, 1)),
        ],
        out_specs=pl.BlockSpec((_BM, 1), lambda i: (i, 0)),
        out_shape=jax.ShapeDtypeStruct((_BC, 1), jnp.float32),
        compiler_params=pltpu.CompilerParams(
            dimension_semantics=("arbitrary",)),
    )(emb3, nump, alph, cbias, w1p, b1, w2, b2, w3, b3, wct, wcb, bc2)


def kernel(categorical_input, numerical_input, emb_tables, cross_alphas,
           cross_bias, W1, b1, W2, b2, W3, b3, Wc, bc):
    tabp = jnp.pad(emb_tables,
                   ((0, 0), (0, 0), (0, 128 - _E))).reshape(_F * _V, 128)
    offs = (jnp.arange(_F, dtype=jnp.int32) * _V)[:, None]
    idxT = categorical_input.astype(jnp.int32).T + offs

    pad = _DP - _D
    nump = jnp.pad(numerical_input, ((0, 0), (0, 128 - _NUM)))
    alph = jnp.pad(cross_alphas[:, :, 0], ((0, 0), (0, pad)))
    cbias = jnp.pad(cross_bias, ((0, 0), (0, pad)))
    w1p = jnp.pad(W1.astype(jnp.bfloat16), ((0, pad), (0, 0)))
    wct = jnp.pad(Wc[:_D], ((0, pad), (0, 0)))
    wcb = Wc[_D:]
    w2b = W2.astype(jnp.bfloat16)
    w3b = W3.astype(jnp.bfloat16)

    gather = _make_sc_gather()
    outs = []
    for c in range(_NCH):
        idx_c = idxT[:, c * _BC:(c + 1) * _BC].reshape(_RC)
        emb3 = gather(tabp, idx_c).reshape(_F, _BC, 128)
        outs.append(_dense_call(
            emb3, lax.dynamic_slice_in_dim(nump, c * _BC, _BC, 0),
            alph, cbias, w1p, b1.reshape(1, _H1),
            w2b, b2.reshape(1, _H2), w3b, b3.reshape(1, _H3),
            wct, wcb, bc.reshape(1, 1)))
    return jnp.concatenate(outs, axis=0)


# speedup vs baseline: 1.0995x; 1.0995x over previous
import functools

import jax
import jax.numpy as jnp
from jax import lax
from jax.experimental import pallas as pl
from jax.experimental.pallas import tpu as pltpu
from jax.experimental.pallas import tpu_sc as plsc

_B = 4096
_F = 26
_V = 1000
_E = 64
_NUM = 13
_L = 3
_D = _F * _E + _NUM
_DP = 1792
_H1, _H2, _H3 = 1024, 512, 256

_NC = 2
_NS = 16
_NW = _NC * _NS
_NCH = 2
_BC = _B // _NCH
_RC = _BC * _F
_RPW = _RC // _NW
_CHUNK = 416
_NCHUNK = _RPW // _CHUNK


@functools.lru_cache(maxsize=1)
def _make_sc_gather():
    mesh = plsc.VectorSubcoreMesh(core_axis_name="c", subcore_axis_name="s")

    @functools.partial(
        pl.kernel,
        mesh=mesh,
        out_type=jax.ShapeDtypeStruct((_RC, 128), jnp.float32),
        scratch_types=[
            pltpu.VMEM((_RPW,), jnp.int32),
            pltpu.VMEM((_CHUNK, 128), jnp.float32),
            pltpu.VMEM((_CHUNK, 128), jnp.float32),
            pltpu.SemaphoreType.DMA,
            pltpu.SemaphoreType.DMA,
        ],
    )
    def _sc_gather(tab_hbm, idx_hbm, out_hbm, idx_v, rows0, rows1, sem0, sem1):
        wid = lax.axis_index("s") * _NC + lax.axis_index("c")
        base = wid * _RPW
        pltpu.sync_copy(idx_hbm.at[pl.ds(base, _RPW)], idx_v)
        bufs = (rows0, rows1)
        sems = (sem0, sem1)
        cps = [None, None]
        cps[0] = pltpu.async_copy(tab_hbm.at[idx_v.at[pl.ds(0, _CHUNK)]],
                                  rows0, sem0)
        for c in range(_NCHUNK):
            nxt = (c + 1) % 2
            if c + 1 < _NCHUNK:
                cps[nxt] = pltpu.async_copy(
                    tab_hbm.at[idx_v.at[pl.ds((c + 1) * _CHUNK, _CHUNK)]],
                    bufs[nxt], sems[nxt])
            cps[c % 2].wait()
            pltpu.sync_copy(bufs[c % 2],
                            out_hbm.at[pl.ds(base + c * _CHUNK, _CHUNK)])

    return _sc_gather


_BM = 256


def _dense_block(emb_ref, num_ref, alph_ref, cbias_ref, w1_ref, b1_ref,
                 w2_ref, b2_ref, w3_ref, b3_ref, wct_ref, wcb_ref, bc_ref,
                 out_ref):
    pairs = []
    for p in range(_F // 2):
        a = emb_ref[2 * p]
        b = jnp.roll(emb_ref[2 * p + 1], 64, axis=1)
        pairs.append(a + b)
    x = jnp.concatenate(pairs + [num_ref[...]], axis=1)
    xb = x.astype(jnp.bfloat16)
    cross = x
    for l in range(_L):
        s = jnp.sum(cross * alph_ref[l][None, :], axis=1, keepdims=True)
        cross = cross * (1.0 + s) + cbias_ref[l][None, :]
    h = jnp.dot(xb, w1_ref[...], preferred_element_type=jnp.float32)
    h = jnp.maximum(h + b1_ref[...], 0.0)
    h = jnp.dot(h.astype(jnp.bfloat16), w2_ref[...],
                preferred_element_type=jnp.float32)
    h = jnp.maximum(h + b2_ref[...], 0.0)
    h = (jnp.dot(h.astype(jnp.bfloat16), w3_ref[...],
                 preferred_element_type=jnp.float32) + b3_ref[...])
    z = (jnp.dot(cross, wct_ref[...], preferred_element_type=jnp.float32)
         + jnp.dot(h, wcb_ref[...], preferred_element_type=jnp.float32)
         + bc_ref[0, 0])
    out_ref[...] = jax.nn.sigmoid(z)


def _dense_call(emb3, nump, alph, cbias, w1p, b1, w2, b2, w3, b3, wct, wcb, bc2):
    nblk = _BC // _BM
    full = lambda shape: pl.BlockSpec(shape, lambda i: (0,) * len(shape))
    return pl.pallas_call(
        _dense_block,
        grid=(nblk,),
        in_specs=[
            pl.BlockSpec((_F, _BM, 128), lambda i: (0, i, 0)),
            pl.BlockSpec((_BM, 128), lambda i: (i, 0)),
            full((_L, _DP)),
            full((_L, _DP)),
            full((_DP, _H1)),
            full((1, _H1)),
            full((_H1, _H2)),
            full((1, _H2)),
            full((_H2, _H3)),
            full((1, _H3)),
            full((_DP, 1)),
            full((_H3, 1)),
            full((1, 1)),
        ],
        out_specs=pl.BlockSpec((_BM, 1), lambda i: (i, 0)),
        out_shape=jax.ShapeDtypeStruct((_BC, 1), jnp.float32),
        compiler_params=pltpu.CompilerParams(
            dimension_semantics=("arbitrary",)),
    )(emb3, nump, alph, cbias, w1p, b1, w2, b2, w3, b3, wct, wcb, bc2)


def kernel(categorical_input, numerical_input, emb_tables, cross_alphas,
           cross_bias, W1, b1, W2, b2, W3, b3, Wc, bc):
    tabp = jnp.pad(emb_tables,
                   ((0, 0), (0, 0), (0, 128 - _E))).reshape(_F * _V, 128)
    offs = (jnp.arange(_F, dtype=jnp.int32) * _V)[:, None]
    idxT = categorical_input.astype(jnp.int32).T + offs

    pad = _DP - _D
    nump = jnp.pad(numerical_input, ((0, 0), (0, 128 - _NUM)))
    alph = jnp.pad(cross_alphas[:, :, 0], ((0, 0), (0, pad)))
    cbias = jnp.pad(cross_bias, ((0, 0), (0, pad)))
    w1p = jnp.pad(W1.astype(jnp.bfloat16), ((0, pad), (0, 0)))
    wct = jnp.pad(Wc[:_D], ((0, pad), (0, 0)))
    wcb = Wc[_D:]
    w2b = W2.astype(jnp.bfloat16)
    w3b = W3.astype(jnp.bfloat16)

    gather = _make_sc_gather()
    outs = []
    for c in range(_NCH):
        idx_c = idxT[:, c * _BC:(c + 1) * _BC].reshape(_RC)
        emb3 = gather(tabp, idx_c).reshape(_F, _BC, 128)
        outs.append(_dense_call(
            emb3, lax.dynamic_slice_in_dim(nump, c * _BC, _BC, 0),
            alph, cbias, w1p, b1.reshape(1, _H1),
            w2b, b2.reshape(1, _H2), w3b, b3.reshape(1, _H3),
            wct, wcb, bc.reshape(1, 1)))
    return jnp.concatenate(outs, axis=0)


# speedup vs baseline: 1.1528x; 1.0485x over previous
import functools

import jax
import jax.numpy as jnp
from jax import lax
from jax.experimental import pallas as pl
from jax.experimental.pallas import tpu as pltpu
from jax.experimental.pallas import tpu_sc as plsc

_B = 4096
_F = 26
_V = 1000
_E = 64
_NUM = 13
_L = 3
_D = _F * _E + _NUM
_DP = 1792
_H1, _H2, _H3 = 1024, 512, 256

_NC = 2
_NS = 16
_NW = _NC * _NS
_NCH = 2
_BC = _B // _NCH
_P = _F // 2
_RC = _BC * _P
_RPW = _RC // _NW
_CHUNK = 208
_NCHUNK = _RPW // _CHUNK


@functools.lru_cache(maxsize=1)
def _make_sc_gather():
    mesh = plsc.VectorSubcoreMesh(core_axis_name="c", subcore_axis_name="s")

    @functools.partial(
        pl.kernel,
        mesh=mesh,
        out_type=jax.ShapeDtypeStruct((_RC, 128), jnp.float32),
        scratch_types=[
            pltpu.VMEM((_RPW,), jnp.int32),
            pltpu.VMEM((_RPW,), jnp.int32),
            pltpu.VMEM((_CHUNK, 128), jnp.float32),
            pltpu.VMEM((_CHUNK, 128), jnp.float32),
            pltpu.VMEM((_CHUNK, 128), jnp.float32),
            pltpu.VMEM((_CHUNK, 128), jnp.float32),
            pltpu.SemaphoreType.DMA,
            pltpu.SemaphoreType.DMA,
        ],
    )
    def _sc_gather(tab_hbm, idxa_hbm, idxb_hbm, out_hbm,
                   idxa_v, idxb_v, a0, b0, a1, b1, sem0, sem1):
        wid = lax.axis_index("s") * _NC + lax.axis_index("c")
        base = wid * _RPW
        pltpu.sync_copy(idxa_hbm.at[pl.ds(base, _RPW)], idxa_v)
        pltpu.sync_copy(idxb_hbm.at[pl.ds(base, _RPW)], idxb_v)
        abufs = (a0, a1)
        bbufs = (b0, b1)
        sems = (sem0, sem1)
        cps = [None, None]

        def start(c):
            slot = c % 2
            cpa = pltpu.async_copy(
                tab_hbm.at[idxa_v.at[pl.ds(c * _CHUNK, _CHUNK)]],
                abufs[slot], sems[slot])
            cpb = pltpu.async_copy(
                tab_hbm.at[idxb_v.at[pl.ds(c * _CHUNK, _CHUNK)]],
                bbufs[slot], sems[slot])
            cps[slot] = (cpa, cpb)

        start(0)
        for c in range(_NCHUNK):
            slot = c % 2
            if c + 1 < _NCHUNK:
                start(c + 1)
            cps[slot][0].wait()
            cps[slot][1].wait()
            ab = abufs[slot]
            bb = bbufs[slot]

            @pl.loop(0, _CHUNK, unroll=8)
            def _(r):
                for k in range(4):
                    ab[r, pl.ds(64 + 16 * k, 16)] = bb[r, pl.ds(16 * k, 16)]

            pltpu.sync_copy(ab, out_hbm.at[pl.ds(base + c * _CHUNK, _CHUNK)])

    return _sc_gather


_BM = 256


def _dense_block(emb_ref, num_ref, alph_ref, cbias_ref, w1_ref, b1_ref,
                 w2_ref, b2_ref, w3_ref, b3_ref, wct_ref, wcb_ref, bc_ref,
                 out_ref):
    slabs = [emb_ref[p] for p in range(_P)]
    x = jnp.concatenate(slabs + [num_ref[...]], axis=1)
    xb = x.astype(jnp.bfloat16)
    cross = x
    for l in range(_L):
        s = jnp.sum(cross * alph_ref[l][None, :], axis=1, keepdims=True)
        cross = cross * (1.0 + s) + cbias_ref[l][None, :]
    h = jnp.dot(xb, w1_ref[...], preferred_element_type=jnp.float32)
    h = jnp.maximum(h + b1_ref[...], 0.0)
    h = jnp.dot(h.astype(jnp.bfloat16), w2_ref[...],
                preferred_element_type=jnp.float32)
    h = jnp.maximum(h + b2_ref[...], 0.0)
    h = (jnp.dot(h.astype(jnp.bfloat16), w3_ref[...],
                 preferred_element_type=jnp.float32) + b3_ref[...])
    z = (jnp.dot(cross, wct_ref[...], preferred_element_type=jnp.float32)
         + jnp.dot(h, wcb_ref[...], preferred_element_type=jnp.float32)
         + bc_ref[0, 0])
    out_ref[...] = jax.nn.sigmoid(z)


def _dense_call(emb3, nump, alph, cbias, w1p, b1, w2, b2, w3, b3, wct, wcb, bc2):
    nblk = _BC // _BM
    full = lambda shape: pl.BlockSpec(shape, lambda i: (0,) * len(shape))
    return pl.pallas_call(
        _dense_block,
        grid=(nblk,),
        in_specs=[
            pl.BlockSpec((_P, _BM, 128), lambda i: (0, i, 0)),
            pl.BlockSpec((_BM, 128), lambda i: (i, 0)),
            full((_L, _DP)),
            full((_L, _DP)),
            full((_DP, _H1)),
            full((1, _H1)),
            full((_H1, _H2)),
            full((1, _H2)),
            full((_H2, _H3)),
            full((1, _H3)),
            full((_DP, 1)),
            full((_H3, 1)),
            full((1, 1)),
        ],
        out_specs=pl.BlockSpec((_BM, 1), lambda i: (i, 0)),
        out_shape=jax.ShapeDtypeStruct((_BC, 1), jnp.float32),
        compiler_params=pltpu.CompilerParams(
            dimension_semantics=("arbitrary",)),
    )(emb3, nump, alph, cbias, w1p, b1, w2, b2, w3, b3, wct, wcb, bc2)


def kernel(categorical_input, numerical_input, emb_tables, cross_alphas,
           cross_bias, W1, b1, W2, b2, W3, b3, Wc, bc):
    tabp = jnp.pad(emb_tables,
                   ((0, 0), (0, 0), (0, 128 - _E))).reshape(_F * _V, 128)
    offs = (jnp.arange(_F, dtype=jnp.int32) * _V)[:, None]
    idxT = categorical_input.astype(jnp.int32).T + offs
    idxA = idxT[0::2]
    idxB = idxT[1::2]

    pad = _DP - _D
    nump = jnp.pad(numerical_input, ((0, 0), (0, 128 - _NUM)))
    alph = jnp.pad(cross_alphas[:, :, 0], ((0, 0), (0, pad)))
    cbias = jnp.pad(cross_bias, ((0, 0), (0, pad)))
    w1p = jnp.pad(W1.astype(jnp.bfloat16), ((0, pad), (0, 0)))
    wct = jnp.pad(Wc[:_D], ((0, pad), (0, 0)))
    wcb = Wc[_D:]
    w2b = W2.astype(jnp.bfloat16)
    w3b = W3.astype(jnp.bfloat16)

    gather = _make_sc_gather()
    outs = []
    for c in range(_NCH):
        ia = idxA[:, c * _BC:(c + 1) * _BC].reshape(_RC)
        ib = idxB[:, c * _BC:(c + 1) * _BC].reshape(_RC)
        emb3 = gather(tabp, ia, ib).reshape(_P, _BC, 128)
        outs.append(_dense_call(
            emb3, lax.dynamic_slice_in_dim(nump, c * _BC, _BC, 0),
            alph, cbias, w1p, b1.reshape(1, _H1),
            w2b, b2.reshape(1, _H2), w3b, b3.reshape(1, _H3),
            wct, wcb, bc.reshape(1, 1)))
    return jnp.concatenate(outs, axis=0)


# speedup vs baseline: 1.1714x; 1.0162x over previous
import functools

import jax
import jax.numpy as jnp
from jax import lax
from jax.experimental import pallas as pl
from jax.experimental.pallas import tpu as pltpu
from jax.experimental.pallas import tpu_sc as plsc

_B = 4096
_F = 26
_V = 1000
_E = 64
_NUM = 13
_L = 3
_D = _F * _E + _NUM
_DP = 1792
_H1, _H2, _H3 = 1024, 512, 256

_NC = 2
_NS = 16
_NW = _NC * _NS
_NCH = 2
_BC = _B // _NCH
_P = _F // 2
_RC = _BC * _P
_RPW = _RC // _NW
_CHUNK = 208
_NCHUNK = _RPW // _CHUNK


@functools.lru_cache(maxsize=1)
def _make_sc_gather():
    mesh = plsc.VectorSubcoreMesh(core_axis_name="c", subcore_axis_name="s")

    @functools.partial(
        pl.kernel,
        mesh=mesh,
        out_type=jax.ShapeDtypeStruct((_RC, 128), jnp.float32),
        scratch_types=[
            pltpu.VMEM((_RPW,), jnp.int32),
            pltpu.VMEM((_RPW,), jnp.int32),
            pltpu.VMEM((_CHUNK, 128), jnp.float32),
            pltpu.VMEM((_CHUNK, 128), jnp.float32),
            pltpu.VMEM((_CHUNK, 128), jnp.float32),
            pltpu.VMEM((_CHUNK, 128), jnp.float32),
            pltpu.SemaphoreType.DMA,
            pltpu.SemaphoreType.DMA,
        ],
    )
    def _sc_gather(tab_hbm, idxa_hbm, idxb_hbm, out_hbm,
                   idxa_v, idxb_v, a0, b0, a1, b1, sem0, sem1):
        wid = lax.axis_index("s") * _NC + lax.axis_index("c")
        base = wid * _RPW
        pltpu.sync_copy(idxa_hbm.at[pl.ds(base, _RPW)], idxa_v)
        pltpu.sync_copy(idxb_hbm.at[pl.ds(base, _RPW)], idxb_v)
        abufs = (a0, a1)
        bbufs = (b0, b1)
        sems = (sem0, sem1)
        cps = [None, None]

        def start(c):
            slot = c % 2
            cpa = pltpu.async_copy(
                tab_hbm.at[idxa_v.at[pl.ds(c * _CHUNK, _CHUNK)]],
                abufs[slot], sems[slot])
            cpb = pltpu.async_copy(
                tab_hbm.at[idxb_v.at[pl.ds(c * _CHUNK, _CHUNK)]],
                bbufs[slot], sems[slot])
            cps[slot] = (cpa, cpb)

        start(0)
        for c in range(_NCHUNK):
            slot = c % 2
            if c + 1 < _NCHUNK:
                start(c + 1)
            cps[slot][0].wait()
            cps[slot][1].wait()
            ab = abufs[slot]
            bb = bbufs[slot]

            @pl.loop(0, _CHUNK, unroll=8)
            def _(r):
                for k in range(4):
                    ab[r, pl.ds(64 + 16 * k, 16)] = bb[r, pl.ds(16 * k, 16)]

            pltpu.sync_copy(ab, out_hbm.at[pl.ds(base + c * _CHUNK, _CHUNK)])

    return _sc_gather


_BM = 512


def _dense_block(emb_ref, num_ref, alph_ref, cbias_ref, w1_ref, b1_ref,
                 w2_ref, b2_ref, w3_ref, b3_ref, wct_ref, wcb_ref, bc_ref,
                 out_ref):
    slabs = [emb_ref[p] for p in range(_P)]
    x = jnp.concatenate(slabs + [num_ref[...]], axis=1)
    xb = x.astype(jnp.bfloat16)
    cross = x
    for l in range(_L):
        s = jnp.sum(cross * alph_ref[l][None, :], axis=1, keepdims=True)
        cross = cross * (1.0 + s) + cbias_ref[l][None, :]
    h = jnp.dot(xb, w1_ref[...], preferred_element_type=jnp.float32)
    h = jnp.maximum(h + b1_ref[...], 0.0)
    h = jnp.dot(h.astype(jnp.bfloat16), w2_ref[...],
                preferred_element_type=jnp.float32)
    h = jnp.maximum(h + b2_ref[...], 0.0)
    h = (jnp.dot(h.astype(jnp.bfloat16), w3_ref[...],
                 preferred_element_type=jnp.float32) + b3_ref[...])
    z = (jnp.dot(cross, wct_ref[...], preferred_element_type=jnp.float32)
         + jnp.dot(h, wcb_ref[...], preferred_element_type=jnp.float32)
         + bc_ref[0, 0])
    out_ref[...] = jax.nn.sigmoid(z)


def _dense_call(emb3, nump, alph, cbias, w1p, b1, w2, b2, w3, b3, wct, wcb, bc2):
    nblk = _BC // _BM
    full = lambda shape: pl.BlockSpec(shape, lambda i: (0,) * len(shape))
    return pl.pallas_call(
        _dense_block,
        grid=(nblk,),
        in_specs=[
            pl.BlockSpec((_P, _BM, 128), lambda i: (0, i, 0)),
            pl.BlockSpec((_BM, 128), lambda i: (i, 0)),
            full((_L, _DP)),
            full((_L, _DP)),
            full((_DP, _H1)),
            full((1, _H1)),
            full((_H1, _H2)),
            full((1, _H2)),
            full((_H2, _H3)),
            full((1, _H3)),
            full((_DP, 1)),
            full((_H3, 1)),
            full((1, 1)),
        ],
        out_specs=pl.BlockSpec((_BM, 1), lambda i: (i, 0)),
        out_shape=jax.ShapeDtypeStruct((_BC, 1), jnp.float32),
        compiler_params=pltpu.CompilerParams(
            dimension_semantics=("arbitrary",)),
    )(emb3, nump, alph, cbias, w1p, b1, w2, b2, w3, b3, wct, wcb, bc2)


def kernel(categorical_input, numerical_input, emb_tables, cross_alphas,
           cross_bias, W1, b1, W2, b2, W3, b3, Wc, bc):
    tabp = jnp.pad(emb_tables,
                   ((0, 0), (0, 0), (0, 128 - _E))).reshape(_F * _V, 128)
    offs = (jnp.arange(_F, dtype=jnp.int32) * _V)[:, None]
    idxT = categorical_input.astype(jnp.int32).T + offs
    idxA = idxT[0::2]
    idxB = idxT[1::2]

    pad = _DP - _D
    nump = jnp.pad(numerical_input, ((0, 0), (0, 128 - _NUM)))
    alph = jnp.pad(cross_alphas[:, :, 0], ((0, 0), (0, pad)))
    cbias = jnp.pad(cross_bias, ((0, 0), (0, pad)))
    w1p = jnp.pad(W1.astype(jnp.bfloat16), ((0, pad), (0, 0)))
    wct = jnp.pad(Wc[:_D], ((0, pad), (0, 0)))
    wcb = Wc[_D:]
    w2b = W2.astype(jnp.bfloat16)
    w3b = W3.astype(jnp.bfloat16)

    gather = _make_sc_gather()
    outs = []
    for c in range(_NCH):
        ia = idxA[:, c * _BC:(c + 1) * _BC].reshape(_RC)
        ib = idxB[:, c * _BC:(c + 1) * _BC].reshape(_RC)
        emb3 = gather(tabp, ia, ib).reshape(_P, _BC, 128)
        outs.append(_dense_call(
            emb3, lax.dynamic_slice_in_dim(nump, c * _BC, _BC, 0),
            alph, cbias, w1p, b1.reshape(1, _H1),
            w2b, b2.reshape(1, _H2), w3b, b3.reshape(1, _H3),
            wct, wcb, bc.reshape(1, 1)))
    return jnp.concatenate(outs, axis=0)


# speedup vs baseline: 1.1820x; 1.0090x over previous
import functools

import jax
import jax.numpy as jnp
from jax import lax
from jax.experimental import pallas as pl
from jax.experimental.pallas import tpu as pltpu
from jax.experimental.pallas import tpu_sc as plsc

_B = 4096
_F = 26
_V = 1000
_E = 64
_NUM = 13
_L = 3
_D = _F * _E + _NUM
_DP = 1792
_H1, _H2, _H3 = 1024, 512, 256

_NC = 2
_NS = 16
_NW = _NC * _NS
_NCH = 2
_BC = _B // _NCH
_P = _F // 2
_RC = _BC * _P
_RPW = _RC // _NW
_CHUNK = 208
_NCHUNK = _RPW // _CHUNK


@functools.lru_cache(maxsize=1)
def _make_sc_gather():
    mesh = plsc.VectorSubcoreMesh(core_axis_name="c", subcore_axis_name="s")

    @functools.partial(
        pl.kernel,
        mesh=mesh,
        out_type=jax.ShapeDtypeStruct((_RC, 128), jnp.float32),
        scratch_types=[
            pltpu.VMEM((_RPW,), jnp.int32),
            pltpu.VMEM((_RPW,), jnp.int32),
            pltpu.VMEM((_CHUNK, 128), jnp.float32),
            pltpu.VMEM((_CHUNK, 128), jnp.float32),
            pltpu.VMEM((_CHUNK, 128), jnp.float32),
            pltpu.VMEM((_CHUNK, 128), jnp.float32),
            pltpu.SemaphoreType.DMA,
            pltpu.SemaphoreType.DMA,
            pltpu.SemaphoreType.DMA,
            pltpu.SemaphoreType.DMA,
        ],
    )
    def _sc_gather(tab_hbm, idxa_hbm, idxb_hbm, out_hbm,
                   idxa_v, idxb_v, a0, b0, a1, b1, sem0, sem1, osem0, osem1):
        wid = lax.axis_index("s") * _NC + lax.axis_index("c")
        base = wid * _RPW
        pltpu.sync_copy(idxa_hbm.at[pl.ds(base, _RPW)], idxa_v)
        pltpu.sync_copy(idxb_hbm.at[pl.ds(base, _RPW)], idxb_v)
        abufs = (a0, a1)
        bbufs = (b0, b1)
        sems = (sem0, sem1)
        osems = (osem0, osem1)
        cps = [None, None]
        owr = [None, None]

        def start(c):
            slot = c % 2
            if owr[slot] is not None:
                owr[slot].wait()
            cpa = pltpu.async_copy(
                tab_hbm.at[idxa_v.at[pl.ds(c * _CHUNK, _CHUNK)]],
                abufs[slot], sems[slot])
            cpb = pltpu.async_copy(
                tab_hbm.at[idxb_v.at[pl.ds(c * _CHUNK, _CHUNK)]],
                bbufs[slot], sems[slot])
            cps[slot] = (cpa, cpb)

        start(0)
        for c in range(_NCHUNK):
            slot = c % 2
            if c + 1 < _NCHUNK:
                start(c + 1)
            cps[slot][0].wait()
            cps[slot][1].wait()
            ab = abufs[slot]
            bb = bbufs[slot]

            @pl.loop(0, _CHUNK, unroll=8)
            def _(r):
                for k in range(4):
                    ab[r, pl.ds(64 + 16 * k, 16)] = bb[r, pl.ds(16 * k, 16)]

            owr[slot] = pltpu.async_copy(
                ab, out_hbm.at[pl.ds(base + c * _CHUNK, _CHUNK)], osems[slot])
        owr[0].wait()
        owr[1].wait()

    return _sc_gather


_BM = 512


def _dense_block(emb_ref, num_ref, alph_ref, cbias_ref, w1_ref, b1_ref,
                 w2_ref, b2_ref, w3_ref, b3_ref, wct_ref, wcb_ref, bc_ref,
                 out_ref):
    slabs = [emb_ref[p] for p in range(_P)]
    x = jnp.concatenate(slabs + [num_ref[...]], axis=1)
    xb = x.astype(jnp.bfloat16)
    cross = x
    for l in range(_L):
        s = jnp.sum(cross * alph_ref[l][None, :], axis=1, keepdims=True)
        cross = cross * (1.0 + s) + cbias_ref[l][None, :]
    h = jnp.dot(xb, w1_ref[...], preferred_element_type=jnp.float32)
    h = jnp.maximum(h + b1_ref[...], 0.0)
    h = jnp.dot(h.astype(jnp.bfloat16), w2_ref[...],
                preferred_element_type=jnp.float32)
    h = jnp.maximum(h + b2_ref[...], 0.0)
    h = (jnp.dot(h.astype(jnp.bfloat16), w3_ref[...],
                 preferred_element_type=jnp.float32) + b3_ref[...])
    z = (jnp.dot(cross, wct_ref[...], preferred_element_type=jnp.float32)
         + jnp.dot(h, wcb_ref[...], preferred_element_type=jnp.float32)
         + bc_ref[0, 0])
    out_ref[...] = jax.nn.sigmoid(z)


def _dense_call(emb3, nump, alph, cbias, w1p, b1, w2, b2, w3, b3, wct, wcb, bc2):
    nblk = _BC // _BM
    full = lambda shape: pl.BlockSpec(shape, lambda i: (0,) * len(shape))
    return pl.pallas_call(
        _dense_block,
        grid=(nblk,),
        in_specs=[
            pl.BlockSpec((_P, _BM, 128), lambda i: (0, i, 0)),
            pl.BlockSpec((_BM, 128), lambda i: (i, 0)),
            full((_L, _DP)),
            full((_L, _DP)),
            full((_DP, _H1)),
            full((1, _H1)),
            full((_H1, _H2)),
            full((1, _H2)),
            full((_H2, _H3)),
            full((1, _H3)),
            full((_DP, 1)),
            full((_H3, 1)),
            full((1, 1)),
        ],
        out_specs=pl.BlockSpec((_BM, 1), lambda i: (i, 0)),
        out_shape=jax.ShapeDtypeStruct((_BC, 1), jnp.float32),
        compiler_params=pltpu.CompilerParams(
            dimension_semantics=("arbitrary",)),
    )(emb3, nump, alph, cbias, w1p, b1, w2, b2, w3, b3, wct, wcb, bc2)


def kernel(categorical_input, numerical_input, emb_tables, cross_alphas,
           cross_bias, W1, b1, W2, b2, W3, b3, Wc, bc):
    tabp = jnp.pad(emb_tables,
                   ((0, 0), (0, 0), (0, 128 - _E))).reshape(_F * _V, 128)
    offs = (jnp.arange(_F, dtype=jnp.int32) * _V)[:, None]
    idxT = categorical_input.astype(jnp.int32).T + offs
    idxA = idxT[0::2]
    idxB = idxT[1::2]

    pad = _DP - _D
    nump = jnp.pad(numerical_input, ((0, 0), (0, 128 - _NUM)))
    alph = jnp.pad(cross_alphas[:, :, 0], ((0, 0), (0, pad)))
    cbias = jnp.pad(cross_bias, ((0, 0), (0, pad)))
    w1p = jnp.pad(W1.astype(jnp.bfloat16), ((0, pad), (0, 0)))
    wct = jnp.pad(Wc[:_D], ((0, pad), (0, 0)))
    wcb = Wc[_D:]
    w2b = W2.astype(jnp.bfloat16)
    w3b = W3.astype(jnp.bfloat16)

    gather = _make_sc_gather()
    outs = []
    for c in range(_NCH):
        ia = idxA[:, c * _BC:(c + 1) * _BC].reshape(_RC)
        ib = idxB[:, c * _BC:(c + 1) * _BC].reshape(_RC)
        emb3 = gather(tabp, ia, ib).reshape(_P, _BC, 128)
        outs.append(_dense_call(
            emb3, lax.dynamic_slice_in_dim(nump, c * _BC, _BC, 0),
            alph, cbias, w1p, b1.reshape(1, _H1),
            w2b, b2.reshape(1, _H2), w3b, b3.reshape(1, _H3),
            wct, wcb, bc.reshape(1, 1)))
    return jnp.concatenate(outs, axis=0)
